# baseline (device time: 102336 ns/iter reference)
import jax
import jax.numpy as jnp
from jax import lax
from jax.experimental import pallas as pl
from jax.experimental.pallas import tpu as pltpu

N_DEV = 4
SQ = 2048
D_MODEL = 1024
HQ = 8
DH = 128
QB = 256
WIN = 128
KW = 512
SCALE = 0.08838834764831843
CHUNK = SQ // N_DEV
HALF = D_MODEL // 2
NSTEPS = 2 * (N_DEV - 1)

BF = jnp.bfloat16


def _body(x_ref, wq_ref, k_ref, v_ref, wo_ref, out_ref,
          part_ref, kbf_ref, vbf_ref, ctx_ref,
          comm_r, comm_l, send_r, recv_r, send_l, recv_l):
    my = lax.axis_index("i")
    left = (my - 1) % N_DEV
    right = (my + 1) % N_DEV

    barrier_sem = pltpu.get_barrier_semaphore()
    for nbr in (left, right):
        pl.semaphore_signal(
            barrier_sem, inc=1,
            device_id=(nbr,), device_id_type=pl.DeviceIdType.MESH,
        )
    pl.semaphore_wait(barrier_sem, 2)

    kbf_ref[:, :] = k_ref[:, :].astype(BF)
    vbf_ref[:, :] = v_ref[:, :].astype(BF)

    def compute_chunk(c):
        for sub in range(CHUNK // QB):
            q0 = pl.multiple_of(c * CHUNK + sub * QB, QB)
            lo = pl.multiple_of(jnp.clip(q0 - WIN, 0, SQ - KW), WIN)
            x_blk = x_ref[pl.ds(q0, QB), :].astype(BF)
            q_all = (jnp.dot(x_blk, wq_ref[:, :],
                             preferred_element_type=jnp.float32)
                     * SCALE).astype(BF)
            qi = q0 + lax.broadcasted_iota(jnp.int32, (QB, KW), 0)
            ki = lo + lax.broadcasted_iota(jnp.int32, (QB, KW), 1)
            bias = jnp.where(jnp.abs(qi - ki) <= WIN, 0.0, -1e9).astype(
                jnp.float32)
            for h in range(HQ):
                q_h = q_all[:, h * DH:(h + 1) * DH]
                k_h = kbf_ref[pl.ds(lo, KW), h * DH:(h + 1) * DH]
                v_h = vbf_ref[pl.ds(lo, KW), h * DH:(h + 1) * DH]
                s = lax.dot_general(q_h, k_h, (((1,), (1,)), ((), ())),
                                    preferred_element_type=jnp.float32)
                w = jnp.exp(s + bias)
                rinv = 1.0 / jnp.sum(w, axis=1, keepdims=True)
                ctx = jnp.dot(w.astype(BF), v_h,
                              preferred_element_type=jnp.float32)
                ctx_ref[:, h * DH:(h + 1) * DH] = (ctx * rinv).astype(BF)
            part_ref[pl.ds(q0, QB), :] = jnp.dot(
                ctx_ref[:, :], wo_ref[:, :],
                preferred_element_type=jnp.float32).astype(BF)

    def rows_r(c):
        return part_ref.at[pl.ds(c * CHUNK, CHUNK), pl.ds(0, HALF)]

    def rows_l(c):
        return part_ref.at[pl.ds(c * CHUNK, CHUNK), pl.ds(HALF, HALF)]

    def acc_r(c, slot):
        part_ref[pl.ds(c * CHUNK, CHUNK), pl.ds(0, HALF)] = (
            part_ref[pl.ds(c * CHUNK, CHUNK), pl.ds(0, HALF)]
            + comm_r[slot])

    def acc_l(c, slot):
        part_ref[pl.ds(c * CHUNK, CHUNK), pl.ds(HALF, HALF)] = (
            part_ref[pl.ds(c * CHUNK, CHUNK), pl.ds(HALF, HALF)]
            + comm_l[slot])

    def rdma_r(src, slot, dev):
        return pltpu.make_async_remote_copy(
            src_ref=src, dst_ref=comm_r.at[slot],
            send_sem=send_r.at[slot], recv_sem=recv_r.at[slot],
            device_id=(dev,), device_id_type=pl.DeviceIdType.MESH)

    def rdma_l(src, slot, dev):
        return pltpu.make_async_remote_copy(
            src_ref=src, dst_ref=comm_l.at[slot],
            send_sem=send_l.at[slot], recv_sem=recv_l.at[slot],
            device_id=(dev,), device_id_type=pl.DeviceIdType.MESH)

    c0 = my
    c1 = (my - 1) % N_DEV
    c2 = (my + 1) % N_DEV
    c3 = (my + 2) % N_DEV

    compute_chunk(c0)
    r0 = rdma_r(rows_r(c0), 0, right); r0.start()
    l0 = rdma_l(rows_l(c0), 0, left); l0.start()

    compute_chunk(c1)
    r0.wait_recv()
    acc_r(c1, 0)
    r1 = rdma_r(rows_r(c1), 1, right); r1.start()

    compute_chunk(c2)
    l0.wait_recv()
    acc_l(c2, 0)
    l1 = rdma_l(rows_l(c2), 1, left); l1.start()

    compute_chunk(c3)
    r1.wait_recv()
    acc_r(c3, 1)
    r2 = rdma_r(rows_r(c3), 2, right); r2.start()
    l1.wait_recv()
    acc_l(c3, 1)
    l2 = rdma_l(rows_l(c3), 2, left); l2.start()

    r2.wait_recv()
    acc_r(c2, 2)
    ar0 = rdma_r(rows_r(c2), 3, right); ar0.start()
    l2.wait_recv()
    acc_l(c1, 2)
    al0 = rdma_l(rows_l(c1), 3, left); al0.start()

    out_ref[pl.ds(c2 * CHUNK, CHUNK), pl.ds(0, HALF)] = part_ref[
        pl.ds(c2 * CHUNK, CHUNK), pl.ds(0, HALF)].astype(jnp.float32)
    out_ref[pl.ds(c1 * CHUNK, CHUNK), pl.ds(HALF, HALF)] = part_ref[
        pl.ds(c1 * CHUNK, CHUNK), pl.ds(HALF, HALF)].astype(jnp.float32)

    ar0.wait_recv()
    ar1 = rdma_r(comm_r.at[3], 4, right); ar1.start()
    out_ref[pl.ds(c0 * CHUNK, CHUNK), pl.ds(0, HALF)] = (
        comm_r[3].astype(jnp.float32))
    al0.wait_recv()
    al1 = rdma_l(comm_l.at[3], 4, left); al1.start()
    out_ref[pl.ds(c0 * CHUNK, CHUNK), pl.ds(HALF, HALF)] = (
        comm_l[3].astype(jnp.float32))

    ar1.wait_recv()
    ar2 = rdma_r(comm_r.at[4], 5, right); ar2.start()
    out_ref[pl.ds(c1 * CHUNK, CHUNK), pl.ds(0, HALF)] = (
        comm_r[4].astype(jnp.float32))
    al1.wait_recv()
    al2 = rdma_l(comm_l.at[4], 5, left); al2.start()
    out_ref[pl.ds(c2 * CHUNK, CHUNK), pl.ds(HALF, HALF)] = (
        comm_l[4].astype(jnp.float32))

    ar2.wait_recv()
    out_ref[pl.ds(c3 * CHUNK, CHUNK), pl.ds(0, HALF)] = (
        comm_r[5].astype(jnp.float32))
    al2.wait_recv()
    out_ref[pl.ds(c3 * CHUNK, CHUNK), pl.ds(HALF, HALF)] = (
        comm_l[5].astype(jnp.float32))

    for r in (r0, r1, r2, l0, l1, l2, ar0, ar1, ar2, al0, al1, al2):
        r.wait_send()


def kernel(x, Wq, K_ext, V_ext, Wo):
    my = lax.axis_index("i")
    d_loc = HQ * DH
    wq_loc = lax.dynamic_slice(
        Wq, (0, my * d_loc), (Wq.shape[0], d_loc)).astype(BF)
    wo_loc = lax.dynamic_slice(
        Wo, (my * d_loc, 0), (d_loc, Wo.shape[1])).astype(BF)

    out = pl.pallas_call(
        _body,
        out_shape=jax.ShapeDtypeStruct((SQ, D_MODEL), jnp.float32),
        in_specs=[pl.BlockSpec(memory_space=pltpu.VMEM)] * 5,
        out_specs=pl.BlockSpec(memory_space=pltpu.VMEM),
        scratch_shapes=[
            pltpu.VMEM((SQ, D_MODEL), BF),
            pltpu.VMEM((SQ, HQ * DH), BF),
            pltpu.VMEM((SQ, HQ * DH), BF),
            pltpu.VMEM((QB, HQ * DH), BF),
            pltpu.VMEM((NSTEPS, CHUNK, HALF), BF),
            pltpu.VMEM((NSTEPS, CHUNK, HALF), BF),
            pltpu.SemaphoreType.DMA((NSTEPS,)),
            pltpu.SemaphoreType.DMA((NSTEPS,)),
            pltpu.SemaphoreType.DMA((NSTEPS,)),
            pltpu.SemaphoreType.DMA((NSTEPS,)),
        ],
        compiler_params=pltpu.CompilerParams(
            collective_id=0, vmem_limit_bytes=100 * 1024 * 1024),
    )(x[0], wq_loc, K_ext[0].reshape(SQ, HQ * DH),
      V_ext[0].reshape(SQ, HQ * DH), wo_loc)
    return out[None]


# device time: 97590 ns/iter; 1.0486x vs baseline; 1.0486x over previous
import jax
import jax.numpy as jnp
from jax import lax
from jax.experimental import pallas as pl
from jax.experimental.pallas import tpu as pltpu

N_DEV = 4
SQ = 2048
D_MODEL = 1024
HQ = 8
DH = 128
QB = 256
WIN = 128
KW = 512
SCALE = 0.08838834764831843
CHUNK = SQ // N_DEV
HALF = D_MODEL // 2
NSTEPS = 2 * (N_DEV - 1)

BF = jnp.bfloat16


def _body(x_ref, wq_ref, k_ref, v_ref, wo_ref, out_ref,
          part_ref, ctx_ref,
          comm_r, comm_l, send_r, recv_r, send_l, recv_l):
    my = lax.axis_index("i")
    left = (my - 1) % N_DEV
    right = (my + 1) % N_DEV

    barrier_sem = pltpu.get_barrier_semaphore()
    for nbr in (left, right):
        pl.semaphore_signal(
            barrier_sem, inc=1,
            device_id=(nbr,), device_id_type=pl.DeviceIdType.MESH,
        )
    pl.semaphore_wait(barrier_sem, 2)

    def compute_chunk(c):
        for sub in range(CHUNK // QB):
            q0 = pl.multiple_of(c * CHUNK + sub * QB, QB)
            lo = pl.multiple_of(jnp.clip(q0 - WIN, 0, SQ - KW), WIN)
            x_blk = x_ref[pl.ds(q0, QB), :]
            q_all = (jnp.dot(x_blk, wq_ref[:, :],
                             preferred_element_type=jnp.float32)
                     * SCALE).astype(BF)
            qi = q0 + lax.broadcasted_iota(jnp.int32, (QB, KW), 0)
            ki = lo + lax.broadcasted_iota(jnp.int32, (QB, KW), 1)
            bias = jnp.where(jnp.abs(qi - ki) <= WIN, 0.0, -1e9).astype(
                jnp.float32)
            for h in range(HQ):
                q_h = q_all[:, h * DH:(h + 1) * DH]
                k_h = k_ref[pl.ds(lo, KW), h * DH:(h + 1) * DH]
                v_h = v_ref[pl.ds(lo, KW), h * DH:(h + 1) * DH]
                s = lax.dot_general(q_h, k_h, (((1,), (1,)), ((), ())),
                                    preferred_element_type=jnp.float32)
                w = jnp.exp(s + bias)
                rinv = 1.0 / jnp.sum(w, axis=1, keepdims=True)
                ctx = jnp.dot(w.astype(BF), v_h,
                              preferred_element_type=jnp.float32)
                ctx_ref[:, h * DH:(h + 1) * DH] = (ctx * rinv).astype(BF)
            part_ref[pl.ds(q0, QB), :] = jnp.dot(
                ctx_ref[:, :], wo_ref[:, :],
                preferred_element_type=jnp.float32).astype(BF)

    def rows_r(c):
        return part_ref.at[pl.ds(c * CHUNK, CHUNK), pl.ds(0, HALF)]

    def rows_l(c):
        return part_ref.at[pl.ds(c * CHUNK, CHUNK), pl.ds(HALF, HALF)]

    def acc_r(c, slot):
        part_ref[pl.ds(c * CHUNK, CHUNK), pl.ds(0, HALF)] = (
            part_ref[pl.ds(c * CHUNK, CHUNK), pl.ds(0, HALF)]
            + comm_r[slot])

    def acc_l(c, slot):
        part_ref[pl.ds(c * CHUNK, CHUNK), pl.ds(HALF, HALF)] = (
            part_ref[pl.ds(c * CHUNK, CHUNK), pl.ds(HALF, HALF)]
            + comm_l[slot])

    def rdma_r(src, slot, dev):
        return pltpu.make_async_remote_copy(
            src_ref=src, dst_ref=comm_r.at[slot],
            send_sem=send_r.at[slot], recv_sem=recv_r.at[slot],
            device_id=(dev,), device_id_type=pl.DeviceIdType.MESH)

    def rdma_l(src, slot, dev):
        return pltpu.make_async_remote_copy(
            src_ref=src, dst_ref=comm_l.at[slot],
            send_sem=send_l.at[slot], recv_sem=recv_l.at[slot],
            device_id=(dev,), device_id_type=pl.DeviceIdType.MESH)

    c0 = my
    c1 = (my - 1) % N_DEV
    c2 = (my + 1) % N_DEV
    c3 = (my + 2) % N_DEV

    compute_chunk(c0)
    r0 = rdma_r(rows_r(c0), 0, right); r0.start()
    l0 = rdma_l(rows_l(c0), 0, left); l0.start()

    compute_chunk(c1)
    r0.wait_recv()
    acc_r(c1, 0)
    r1 = rdma_r(rows_r(c1), 1, right); r1.start()

    compute_chunk(c2)
    l0.wait_recv()
    acc_l(c2, 0)
    l1 = rdma_l(rows_l(c2), 1, left); l1.start()

    compute_chunk(c3)
    r1.wait_recv()
    acc_r(c3, 1)
    r2 = rdma_r(rows_r(c3), 2, right); r2.start()
    l1.wait_recv()
    acc_l(c3, 1)
    l2 = rdma_l(rows_l(c3), 2, left); l2.start()

    r2.wait_recv()
    acc_r(c2, 2)
    ar0 = rdma_r(rows_r(c2), 3, right); ar0.start()
    l2.wait_recv()
    acc_l(c1, 2)
    al0 = rdma_l(rows_l(c1), 3, left); al0.start()

    out_ref[pl.ds(c2 * CHUNK, CHUNK), pl.ds(0, HALF)] = part_ref[
        pl.ds(c2 * CHUNK, CHUNK), pl.ds(0, HALF)].astype(jnp.float32)
    out_ref[pl.ds(c1 * CHUNK, CHUNK), pl.ds(HALF, HALF)] = part_ref[
        pl.ds(c1 * CHUNK, CHUNK), pl.ds(HALF, HALF)].astype(jnp.float32)

    ar0.wait_recv()
    ar1 = rdma_r(comm_r.at[3], 4, right); ar1.start()
    out_ref[pl.ds(c0 * CHUNK, CHUNK), pl.ds(0, HALF)] = (
        comm_r[3].astype(jnp.float32))
    al0.wait_recv()
    al1 = rdma_l(comm_l.at[3], 4, left); al1.start()
    out_ref[pl.ds(c0 * CHUNK, CHUNK), pl.ds(HALF, HALF)] = (
        comm_l[3].astype(jnp.float32))

    ar1.wait_recv()
    ar2 = rdma_r(comm_r.at[4], 5, right); ar2.start()
    out_ref[pl.ds(c1 * CHUNK, CHUNK), pl.ds(0, HALF)] = (
        comm_r[4].astype(jnp.float32))
    al1.wait_recv()
    al2 = rdma_l(comm_l.at[4], 5, left); al2.start()
    out_ref[pl.ds(c2 * CHUNK, CHUNK), pl.ds(HALF, HALF)] = (
        comm_l[4].astype(jnp.float32))

    ar2.wait_recv()
    out_ref[pl.ds(c3 * CHUNK, CHUNK), pl.ds(0, HALF)] = (
        comm_r[5].astype(jnp.float32))
    al2.wait_recv()
    out_ref[pl.ds(c3 * CHUNK, CHUNK), pl.ds(HALF, HALF)] = (
        comm_l[5].astype(jnp.float32))

    for r in (r0, r1, r2, l0, l1, l2, ar0, ar1, ar2, al0, al1, al2):
        r.wait_send()


def kernel(x, Wq, K_ext, V_ext, Wo):
    my = lax.axis_index("i")
    d_loc = HQ * DH
    wq_loc = lax.dynamic_slice(
        Wq, (0, my * d_loc), (Wq.shape[0], d_loc)).astype(BF)
    wo_loc = lax.dynamic_slice(
        Wo, (my * d_loc, 0), (d_loc, Wo.shape[1])).astype(BF)

    out = pl.pallas_call(
        _body,
        out_shape=jax.ShapeDtypeStruct((SQ, D_MODEL), jnp.float32),
        in_specs=[pl.BlockSpec(memory_space=pltpu.VMEM)] * 5,
        out_specs=pl.BlockSpec(memory_space=pltpu.VMEM),
        scratch_shapes=[
            pltpu.VMEM((SQ, D_MODEL), BF),
            pltpu.VMEM((QB, HQ * DH), BF),
            pltpu.VMEM((NSTEPS, CHUNK, HALF), BF),
            pltpu.VMEM((NSTEPS, CHUNK, HALF), BF),
            pltpu.SemaphoreType.DMA((NSTEPS,)),
            pltpu.SemaphoreType.DMA((NSTEPS,)),
            pltpu.SemaphoreType.DMA((NSTEPS,)),
            pltpu.SemaphoreType.DMA((NSTEPS,)),
        ],
        compiler_params=pltpu.CompilerParams(
            collective_id=0, vmem_limit_bytes=100 * 1024 * 1024),
    )(x[0].astype(BF), wq_loc,
      K_ext[0].reshape(SQ, HQ * DH).astype(BF),
      V_ext[0].reshape(SQ, HQ * DH).astype(BF), wo_loc)
    return out[None]


# device time: 94275 ns/iter; 1.0855x vs baseline; 1.0352x over previous
import jax
import jax.numpy as jnp
from jax import lax
from jax.experimental import pallas as pl
from jax.experimental.pallas import tpu as pltpu

N_DEV = 4
SQ = 2048
D_MODEL = 1024
HQ = 8
DH = 128
QB = 256
WIN = 128
KW = 512
SCALE = 0.08838834764831843
CHUNK = SQ // N_DEV
HALF = D_MODEL // 2
NSTEPS = 2 * (N_DEV - 1)

BF = jnp.bfloat16


def _body(x_ref, wq_ref, k_ref, v_ref, wo_ref, out_ref,
          part_ref, ctx_ref,
          comm_r, comm_l, send_r, recv_r, send_l, recv_l):
    my = lax.axis_index("i")
    left = (my - 1) % N_DEV
    right = (my + 1) % N_DEV

    barrier_sem = pltpu.get_barrier_semaphore()
    for nbr in (left, right):
        pl.semaphore_signal(
            barrier_sem, inc=1,
            device_id=(nbr,), device_id_type=pl.DeviceIdType.MESH,
        )
    pl.semaphore_wait(barrier_sem, 2)

    def compute_chunk(c):
        for sub in range(CHUNK // QB):
            q0 = pl.multiple_of(c * CHUNK + sub * QB, QB)
            lo = pl.multiple_of(jnp.clip(q0 - WIN, 0, SQ - KW), WIN)
            x_blk = x_ref[pl.ds(q0, QB), :]
            q_all = (jnp.dot(x_blk, wq_ref[:, :],
                             preferred_element_type=jnp.float32)
                     * SCALE).astype(BF)
            qi = q0 + lax.broadcasted_iota(jnp.int32, (QB, KW), 0)
            ki = lo + lax.broadcasted_iota(jnp.int32, (QB, KW), 1)
            bias = jnp.where(jnp.abs(qi - ki) <= WIN, 0.0, -1e9).astype(
                jnp.float32)
            for h in range(HQ):
                q_h = q_all[:, h * DH:(h + 1) * DH]
                k_h = k_ref[pl.ds(lo, KW), h * DH:(h + 1) * DH]
                v_h = v_ref[pl.ds(lo, KW), h * DH:(h + 1) * DH]
                s = lax.dot_general(q_h, k_h, (((1,), (1,)), ((), ())),
                                    preferred_element_type=jnp.float32)
                w = jnp.exp(s + bias)
                rinv = 1.0 / jnp.sum(w, axis=1, keepdims=True)
                ctx = jnp.dot(w.astype(BF), v_h,
                              preferred_element_type=jnp.float32)
                ctx_ref[:, h * DH:(h + 1) * DH] = (ctx * rinv).astype(BF)
            part_ref[pl.ds(q0, QB), :] = jnp.dot(
                ctx_ref[:, :], wo_ref[:, :],
                preferred_element_type=jnp.float32).astype(BF)

    def rows_r(c):
        return part_ref.at[pl.ds(c * CHUNK, CHUNK), pl.ds(0, HALF)]

    def rows_l(c):
        return part_ref.at[pl.ds(c * CHUNK, CHUNK), pl.ds(HALF, HALF)]

    def acc_r(c, slot):
        part_ref[pl.ds(c * CHUNK, CHUNK), pl.ds(0, HALF)] = (
            part_ref[pl.ds(c * CHUNK, CHUNK), pl.ds(0, HALF)]
            + comm_r[slot])

    def acc_l(c, slot):
        part_ref[pl.ds(c * CHUNK, CHUNK), pl.ds(HALF, HALF)] = (
            part_ref[pl.ds(c * CHUNK, CHUNK), pl.ds(HALF, HALF)]
            + comm_l[slot])

    def rdma_r(src, slot, dev):
        return pltpu.make_async_remote_copy(
            src_ref=src, dst_ref=comm_r.at[slot],
            send_sem=send_r.at[slot], recv_sem=recv_r.at[slot],
            device_id=(dev,), device_id_type=pl.DeviceIdType.MESH)

    def rdma_l(src, slot, dev):
        return pltpu.make_async_remote_copy(
            src_ref=src, dst_ref=comm_l.at[slot],
            send_sem=send_l.at[slot], recv_sem=recv_l.at[slot],
            device_id=(dev,), device_id_type=pl.DeviceIdType.MESH)

    c0 = my
    c1 = (my - 1) % N_DEV
    c2 = (my + 1) % N_DEV
    c3 = (my + 2) % N_DEV

    compute_chunk(c0)
    r0 = rdma_r(rows_r(c0), 0, right); r0.start()
    l0 = rdma_l(rows_l(c0), 0, left); l0.start()

    compute_chunk(c1)
    r0.wait_recv()
    acc_r(c1, 0)
    r1 = rdma_r(rows_r(c1), 1, right); r1.start()

    compute_chunk(c2)
    l0.wait_recv()
    acc_l(c2, 0)
    l1 = rdma_l(rows_l(c2), 1, left); l1.start()

    compute_chunk(c3)
    r1.wait_recv()
    acc_r(c3, 1)
    r2 = rdma_r(rows_r(c3), 2, right); r2.start()
    l1.wait_recv()
    acc_l(c3, 1)
    l2 = rdma_l(rows_l(c3), 2, left); l2.start()

    r2.wait_recv()
    acc_r(c2, 2)
    ar0 = rdma_r(rows_r(c2), 3, right); ar0.start()
    l2.wait_recv()
    acc_l(c1, 2)
    al0 = rdma_l(rows_l(c1), 3, left); al0.start()

    ar0b = rdma_r(rows_r(c2), 4, left); ar0b.start()
    al0b = rdma_l(rows_l(c1), 4, right); al0b.start()

    out_ref[pl.ds(c2 * CHUNK, CHUNK), pl.ds(0, HALF)] = part_ref[
        pl.ds(c2 * CHUNK, CHUNK), pl.ds(0, HALF)].astype(jnp.float32)
    out_ref[pl.ds(c1 * CHUNK, CHUNK), pl.ds(HALF, HALF)] = part_ref[
        pl.ds(c1 * CHUNK, CHUNK), pl.ds(HALF, HALF)].astype(jnp.float32)

    ar0.wait_recv()
    ar1 = rdma_r(comm_r.at[3], 5, right); ar1.start()
    out_ref[pl.ds(c0 * CHUNK, CHUNK), pl.ds(0, HALF)] = (
        comm_r[3].astype(jnp.float32))
    al0.wait_recv()
    al1 = rdma_l(comm_l.at[3], 5, left); al1.start()
    out_ref[pl.ds(c0 * CHUNK, CHUNK), pl.ds(HALF, HALF)] = (
        comm_l[3].astype(jnp.float32))

    ar0b.wait_recv()
    out_ref[pl.ds(c3 * CHUNK, CHUNK), pl.ds(0, HALF)] = (
        comm_r[4].astype(jnp.float32))
    al0b.wait_recv()
    out_ref[pl.ds(c3 * CHUNK, CHUNK), pl.ds(HALF, HALF)] = (
        comm_l[4].astype(jnp.float32))

    ar1.wait_recv()
    out_ref[pl.ds(c1 * CHUNK, CHUNK), pl.ds(0, HALF)] = (
        comm_r[5].astype(jnp.float32))
    al1.wait_recv()
    out_ref[pl.ds(c2 * CHUNK, CHUNK), pl.ds(HALF, HALF)] = (
        comm_l[5].astype(jnp.float32))

    for r in (r0, r1, r2, l0, l1, l2, ar0, ar0b, ar1, al0, al0b, al1):
        r.wait_send()


def kernel(x, Wq, K_ext, V_ext, Wo):
    my = lax.axis_index("i")
    d_loc = HQ * DH
    wq_loc = lax.dynamic_slice(
        Wq, (0, my * d_loc), (Wq.shape[0], d_loc)).astype(BF)
    wo_loc = lax.dynamic_slice(
        Wo, (my * d_loc, 0), (d_loc, Wo.shape[1])).astype(BF)

    out = pl.pallas_call(
        _body,
        out_shape=jax.ShapeDtypeStruct((SQ, D_MODEL), jnp.float32),
        in_specs=[pl.BlockSpec(memory_space=pltpu.VMEM)] * 5,
        out_specs=pl.BlockSpec(memory_space=pltpu.VMEM),
        scratch_shapes=[
            pltpu.VMEM((SQ, D_MODEL), BF),
            pltpu.VMEM((QB, HQ * DH), BF),
            pltpu.VMEM((NSTEPS, CHUNK, HALF), BF),
            pltpu.VMEM((NSTEPS, CHUNK, HALF), BF),
            pltpu.SemaphoreType.DMA((NSTEPS,)),
            pltpu.SemaphoreType.DMA((NSTEPS,)),
            pltpu.SemaphoreType.DMA((NSTEPS,)),
            pltpu.SemaphoreType.DMA((NSTEPS,)),
        ],
        compiler_params=pltpu.CompilerParams(
            collective_id=0, vmem_limit_bytes=100 * 1024 * 1024),
    )(x[0].astype(BF), wq_loc,
      K_ext[0].reshape(SQ, HQ * DH).astype(BF),
      V_ext[0].reshape(SQ, HQ * DH).astype(BF), wo_loc)
    return out[None]


# device time: 90652 ns/iter; 1.1289x vs baseline; 1.0400x over previous
import jax
import jax.numpy as jnp
from jax import lax
from jax.experimental import pallas as pl
from jax.experimental.pallas import tpu as pltpu

N_DEV = 4
SQ = 2048
D_MODEL = 1024
HQ = 8
DH = 128
QB = 256
WIN = 128
KW = 512
SCALE = 0.08838834764831843
CHUNK = SQ // N_DEV
HALF = D_MODEL // 2
NSTEPS = 2 * (N_DEV - 1)

BF = jnp.bfloat16


def _body(x_ref, wq_ref, k_ref, v_ref, wo_ref, out_ref,
          part_ref, ctx_ref, wqbf_ref, wobf_ref, wstage_ref,
          comm_r, comm_l, send_r, recv_r, send_l, recv_l, wdma_sem):
    my = lax.axis_index("i")
    left = (my - 1) % N_DEV
    right = (my + 1) % N_DEV
    d0 = pl.multiple_of(my * (HQ * DH), HQ * DH)

    wq_dma = pltpu.make_async_copy(
        wq_ref.at[:, pl.ds(d0, HQ * DH)], wstage_ref, wdma_sem)
    wq_dma.start()

    barrier_sem = pltpu.get_barrier_semaphore()
    for nbr in (left, right):
        pl.semaphore_signal(
            barrier_sem, inc=1,
            device_id=(nbr,), device_id_type=pl.DeviceIdType.MESH,
        )
    pl.semaphore_wait(barrier_sem, 2)

    wq_dma.wait()
    wqbf_ref[:, :] = wstage_ref[:, :].astype(BF)
    wo_dma = pltpu.make_async_copy(
        wo_ref.at[pl.ds(d0, HQ * DH), :], wstage_ref, wdma_sem)
    wo_dma.start()
    wo_dma.wait()
    wobf_ref[:, :] = wstage_ref[:, :].astype(BF)

    def compute_chunk(c):
        for sub in range(CHUNK // QB):
            q0 = pl.multiple_of(c * CHUNK + sub * QB, QB)
            lo = pl.multiple_of(jnp.clip(q0 - WIN, 0, SQ - KW), WIN)
            x_blk = x_ref[pl.ds(q0, QB), :]
            q_all = (jnp.dot(x_blk, wqbf_ref[:, :],
                             preferred_element_type=jnp.float32)
                     * SCALE).astype(BF)
            qi = q0 + lax.broadcasted_iota(jnp.int32, (QB, KW), 0)
            ki = lo + lax.broadcasted_iota(jnp.int32, (QB, KW), 1)
            bias = jnp.where(jnp.abs(qi - ki) <= WIN, 0.0, -1e9).astype(
                jnp.float32)
            for h in range(HQ):
                q_h = q_all[:, h * DH:(h + 1) * DH]
                k_h = k_ref[pl.ds(lo, KW), h * DH:(h + 1) * DH]
                v_h = v_ref[pl.ds(lo, KW), h * DH:(h + 1) * DH]
                s = lax.dot_general(q_h, k_h, (((1,), (1,)), ((), ())),
                                    preferred_element_type=jnp.float32)
                w = jnp.exp(s + bias)
                rinv = 1.0 / jnp.sum(w, axis=1, keepdims=True)
                ctx = jnp.dot(w.astype(BF), v_h,
                              preferred_element_type=jnp.float32)
                ctx_ref[:, h * DH:(h + 1) * DH] = (ctx * rinv).astype(BF)
            part_ref[pl.ds(q0, QB), :] = jnp.dot(
                ctx_ref[:, :], wobf_ref[:, :],
                preferred_element_type=jnp.float32).astype(BF)

    def rows_r(c):
        return part_ref.at[pl.ds(c * CHUNK, CHUNK), pl.ds(0, HALF)]

    def rows_l(c):
        return part_ref.at[pl.ds(c * CHUNK, CHUNK), pl.ds(HALF, HALF)]

    def acc_r(c, slot):
        part_ref[pl.ds(c * CHUNK, CHUNK), pl.ds(0, HALF)] = (
            part_ref[pl.ds(c * CHUNK, CHUNK), pl.ds(0, HALF)]
            + comm_r[slot])

    def acc_l(c, slot):
        part_ref[pl.ds(c * CHUNK, CHUNK), pl.ds(HALF, HALF)] = (
            part_ref[pl.ds(c * CHUNK, CHUNK), pl.ds(HALF, HALF)]
            + comm_l[slot])

    def rdma_r(src, slot, dev):
        return pltpu.make_async_remote_copy(
            src_ref=src, dst_ref=comm_r.at[slot],
            send_sem=send_r.at[slot], recv_sem=recv_r.at[slot],
            device_id=(dev,), device_id_type=pl.DeviceIdType.MESH)

    def rdma_l(src, slot, dev):
        return pltpu.make_async_remote_copy(
            src_ref=src, dst_ref=comm_l.at[slot],
            send_sem=send_l.at[slot], recv_sem=recv_l.at[slot],
            device_id=(dev,), device_id_type=pl.DeviceIdType.MESH)

    c0 = my
    c1 = (my - 1) % N_DEV
    c2 = (my + 1) % N_DEV
    c3 = (my + 2) % N_DEV

    compute_chunk(c0)
    r0 = rdma_r(rows_r(c0), 0, right); r0.start()
    l0 = rdma_l(rows_l(c0), 0, left); l0.start()

    compute_chunk(c1)
    r0.wait_recv()
    acc_r(c1, 0)
    r1 = rdma_r(rows_r(c1), 1, right); r1.start()

    compute_chunk(c2)
    l0.wait_recv()
    acc_l(c2, 0)
    l1 = rdma_l(rows_l(c2), 1, left); l1.start()

    compute_chunk(c3)
    r1.wait_recv()
    acc_r(c3, 1)
    r2 = rdma_r(rows_r(c3), 2, right); r2.start()
    l1.wait_recv()
    acc_l(c3, 1)
    l2 = rdma_l(rows_l(c3), 2, left); l2.start()

    r2.wait_recv()
    acc_r(c2, 2)
    ar0 = rdma_r(rows_r(c2), 3, right); ar0.start()
    l2.wait_recv()
    acc_l(c1, 2)
    al0 = rdma_l(rows_l(c1), 3, left); al0.start()

    ar0b = rdma_r(rows_r(c2), 4, left); ar0b.start()
    al0b = rdma_l(rows_l(c1), 4, right); al0b.start()

    out_ref[pl.ds(c2 * CHUNK, CHUNK), pl.ds(0, HALF)] = part_ref[
        pl.ds(c2 * CHUNK, CHUNK), pl.ds(0, HALF)].astype(jnp.float32)
    out_ref[pl.ds(c1 * CHUNK, CHUNK), pl.ds(HALF, HALF)] = part_ref[
        pl.ds(c1 * CHUNK, CHUNK), pl.ds(HALF, HALF)].astype(jnp.float32)

    ar0.wait_recv()
    ar1 = rdma_r(comm_r.at[3], 5, right); ar1.start()
    out_ref[pl.ds(c0 * CHUNK, CHUNK), pl.ds(0, HALF)] = (
        comm_r[3].astype(jnp.float32))
    al0.wait_recv()
    al1 = rdma_l(comm_l.at[3], 5, left); al1.start()
    out_ref[pl.ds(c0 * CHUNK, CHUNK), pl.ds(HALF, HALF)] = (
        comm_l[3].astype(jnp.float32))

    ar0b.wait_recv()
    out_ref[pl.ds(c3 * CHUNK, CHUNK), pl.ds(0, HALF)] = (
        comm_r[4].astype(jnp.float32))
    al0b.wait_recv()
    out_ref[pl.ds(c3 * CHUNK, CHUNK), pl.ds(HALF, HALF)] = (
        comm_l[4].astype(jnp.float32))

    ar1.wait_recv()
    out_ref[pl.ds(c1 * CHUNK, CHUNK), pl.ds(0, HALF)] = (
        comm_r[5].astype(jnp.float32))
    al1.wait_recv()
    out_ref[pl.ds(c2 * CHUNK, CHUNK), pl.ds(HALF, HALF)] = (
        comm_l[5].astype(jnp.float32))

    for r in (r0, r1, r2, l0, l1, l2, ar0, ar0b, ar1, al0, al0b, al1):
        r.wait_send()


def kernel(x, Wq, K_ext, V_ext, Wo):
    out = pl.pallas_call(
        _body,
        out_shape=jax.ShapeDtypeStruct((SQ, D_MODEL), jnp.float32),
        in_specs=[
            pl.BlockSpec(memory_space=pltpu.VMEM),
            pl.BlockSpec(memory_space=pl.ANY),
            pl.BlockSpec(memory_space=pltpu.VMEM),
            pl.BlockSpec(memory_space=pltpu.VMEM),
            pl.BlockSpec(memory_space=pl.ANY),
        ],
        out_specs=pl.BlockSpec(memory_space=pltpu.VMEM),
        scratch_shapes=[
            pltpu.VMEM((SQ, D_MODEL), BF),
            pltpu.VMEM((QB, HQ * DH), BF),
            pltpu.VMEM((D_MODEL, HQ * DH), BF),
            pltpu.VMEM((HQ * DH, D_MODEL), BF),
            pltpu.VMEM((D_MODEL, D_MODEL), jnp.float32),
            pltpu.VMEM((NSTEPS, CHUNK, HALF), BF),
            pltpu.VMEM((NSTEPS, CHUNK, HALF), BF),
            pltpu.SemaphoreType.DMA((NSTEPS,)),
            pltpu.SemaphoreType.DMA((NSTEPS,)),
            pltpu.SemaphoreType.DMA((NSTEPS,)),
            pltpu.SemaphoreType.DMA((NSTEPS,)),
            pltpu.SemaphoreType.DMA,
        ],
        compiler_params=pltpu.CompilerParams(
            collective_id=0, vmem_limit_bytes=100 * 1024 * 1024),
    )(x[0].astype(BF), Wq,
      K_ext[0].reshape(SQ, HQ * DH).astype(BF),
      V_ext[0].reshape(SQ, HQ * DH).astype(BF), Wo)
    return out[None]


# device time: 89423 ns/iter; 1.1444x vs baseline; 1.0137x over previous
import jax
import jax.numpy as jnp
from jax import lax
from jax.experimental import pallas as pl
from jax.experimental.pallas import tpu as pltpu

N_DEV = 4
SQ = 2048
D_MODEL = 1024
HQ = 8
DH = 128
QB = 256
WIN = 128
KW = 512
SCALE = 0.08838834764831843
CHUNK = SQ // N_DEV
HALF = D_MODEL // 2
NSTEPS = 2 * (N_DEV - 1)
NSEM = 10

BF = jnp.bfloat16


def _body(x_ref, wq_ref, k_ref, v_ref, wo_ref, out_ref,
          part_ref, ctx_ref, wqbf_ref, wobf_ref, wstage_ref,
          comm_r, comm_l, send_r, recv_r, send_l, recv_l, wdma_sem):
    my = lax.axis_index("i")
    left = (my - 1) % N_DEV
    right = (my + 1) % N_DEV
    d0 = pl.multiple_of(my * (HQ * DH), HQ * DH)

    wq_dma = pltpu.make_async_copy(
        wq_ref.at[:, pl.ds(d0, HQ * DH)], wstage_ref, wdma_sem)
    wq_dma.start()

    barrier_sem = pltpu.get_barrier_semaphore()
    for nbr in (left, right):
        pl.semaphore_signal(
            barrier_sem, inc=1,
            device_id=(nbr,), device_id_type=pl.DeviceIdType.MESH,
        )
    pl.semaphore_wait(barrier_sem, 2)

    wq_dma.wait()
    wqbf_ref[:, :] = wstage_ref[:, :].astype(BF)
    wo_dma = pltpu.make_async_copy(
        wo_ref.at[pl.ds(d0, HQ * DH), :], wstage_ref, wdma_sem)
    wo_dma.start()
    wo_dma.wait()
    wobf_ref[:, :] = wstage_ref[:, :].astype(BF)

    d_idx = (lax.broadcasted_iota(jnp.int32, (QB, KW), 0)
             - lax.broadcasted_iota(jnp.int32, (QB, KW), 1))

    def compute_block(c, sub):
        if True:
            q0 = pl.multiple_of(c * CHUNK + sub * QB, QB)
            lo = pl.multiple_of(jnp.clip(q0 - WIN, 0, SQ - KW), WIN)
            x_blk = x_ref[pl.ds(q0, QB), :]
            q_all = (jnp.dot(x_blk, wqbf_ref[:, :],
                             preferred_element_type=jnp.float32)
                     * SCALE).astype(BF)
            bias = jnp.where(jnp.abs(d_idx + (q0 - lo)) <= WIN,
                             0.0, -1e9).astype(jnp.float32)
            for h in range(HQ):
                q_h = q_all[:, h * DH:(h + 1) * DH]
                k_h = k_ref[pl.ds(lo, KW), h * DH:(h + 1) * DH]
                v_h = v_ref[pl.ds(lo, KW), h * DH:(h + 1) * DH]
                s = lax.dot_general(q_h, k_h, (((1,), (1,)), ((), ())),
                                    preferred_element_type=jnp.float32)
                w = jnp.exp(s + bias)
                rinv = 1.0 / jnp.sum(w, axis=1, keepdims=True)
                ctx = jnp.dot(w.astype(BF), v_h,
                              preferred_element_type=jnp.float32)
                ctx_ref[:, h * DH:(h + 1) * DH] = (ctx * rinv).astype(BF)
            part_ref[pl.ds(q0, QB), :] = jnp.dot(
                ctx_ref[:, :], wobf_ref[:, :],
                preferred_element_type=jnp.float32).astype(BF)

    def compute_chunk(c):
        compute_block(c, 0)
        compute_block(c, 1)

    def rows_r(c):
        return part_ref.at[pl.ds(c * CHUNK, CHUNK), pl.ds(0, HALF)]

    def rows_l(c):
        return part_ref.at[pl.ds(c * CHUNK, CHUNK), pl.ds(HALF, HALF)]

    def acc_r(c, slot):
        part_ref[pl.ds(c * CHUNK, CHUNK), pl.ds(0, HALF)] = (
            part_ref[pl.ds(c * CHUNK, CHUNK), pl.ds(0, HALF)]
            + comm_r[slot])

    def acc_l(c, slot):
        part_ref[pl.ds(c * CHUNK, CHUNK), pl.ds(HALF, HALF)] = (
            part_ref[pl.ds(c * CHUNK, CHUNK), pl.ds(HALF, HALF)]
            + comm_l[slot])

    def rdma_r(src, slot, dev):
        return pltpu.make_async_remote_copy(
            src_ref=src, dst_ref=comm_r.at[slot],
            send_sem=send_r.at[slot], recv_sem=recv_r.at[slot],
            device_id=(dev,), device_id_type=pl.DeviceIdType.MESH)

    def rdma_l(src, slot, dev):
        return pltpu.make_async_remote_copy(
            src_ref=src, dst_ref=comm_l.at[slot],
            send_sem=send_l.at[slot], recv_sem=recv_l.at[slot],
            device_id=(dev,), device_id_type=pl.DeviceIdType.MESH)

    def sub_rows_r(c, sub):
        return part_ref.at[pl.ds(c * CHUNK + sub * QB, QB), pl.ds(0, HALF)]

    def sub_rows_l(c, sub):
        return part_ref.at[pl.ds(c * CHUNK + sub * QB, QB),
                           pl.ds(HALF, HALF)]

    def acc_r_sub(c, slot, sub):
        part_ref[pl.ds(c * CHUNK + sub * QB, QB), pl.ds(0, HALF)] = (
            part_ref[pl.ds(c * CHUNK + sub * QB, QB), pl.ds(0, HALF)]
            + comm_r[slot, sub * QB:(sub + 1) * QB, :])

    def acc_l_sub(c, slot, sub):
        part_ref[pl.ds(c * CHUNK + sub * QB, QB), pl.ds(HALF, HALF)] = (
            part_ref[pl.ds(c * CHUNK + sub * QB, QB), pl.ds(HALF, HALF)]
            + comm_l[slot, sub * QB:(sub + 1) * QB, :])

    def rdma_rs(src, slot, sub, sem, dev):
        return pltpu.make_async_remote_copy(
            src_ref=src, dst_ref=comm_r.at[slot, pl.ds(sub * QB, QB), :],
            send_sem=send_r.at[sem], recv_sem=recv_r.at[sem],
            device_id=(dev,), device_id_type=pl.DeviceIdType.MESH)

    def rdma_ls(src, slot, sub, sem, dev):
        return pltpu.make_async_remote_copy(
            src_ref=src, dst_ref=comm_l.at[slot, pl.ds(sub * QB, QB), :],
            send_sem=send_l.at[sem], recv_sem=recv_l.at[sem],
            device_id=(dev,), device_id_type=pl.DeviceIdType.MESH)

    c0 = my
    c1 = (my - 1) % N_DEV
    c2 = (my + 1) % N_DEV
    c3 = (my + 2) % N_DEV

    compute_chunk(c0)
    r0 = rdma_r(rows_r(c0), 0, right); r0.start()
    l0 = rdma_l(rows_l(c0), 0, left); l0.start()

    compute_chunk(c1)
    r0.wait_recv()
    acc_r(c1, 0)
    r1 = rdma_r(rows_r(c1), 1, right); r1.start()

    compute_chunk(c2)
    l0.wait_recv()
    acc_l(c2, 0)
    l1 = rdma_l(rows_l(c2), 1, left); l1.start()

    compute_block(c3, 0)
    r1.wait_recv()
    acc_r_sub(c3, 1, 0)
    r2a = rdma_rs(sub_rows_r(c3, 0), 2, 0, 2, right); r2a.start()
    l1.wait_recv()
    acc_l_sub(c3, 1, 0)
    l2a = rdma_ls(sub_rows_l(c3, 0), 2, 0, 2, left); l2a.start()

    compute_block(c3, 1)
    acc_r_sub(c3, 1, 1)
    r2b = rdma_rs(sub_rows_r(c3, 1), 2, 1, 3, right); r2b.start()
    acc_l_sub(c3, 1, 1)
    l2b = rdma_ls(sub_rows_l(c3, 1), 2, 1, 3, left); l2b.start()

    r2a.wait_recv()
    acc_r_sub(c2, 2, 0)
    ar_sa = rdma_rs(sub_rows_r(c2, 0), 3, 0, 4, right); ar_sa.start()
    ar_pa = rdma_rs(sub_rows_r(c2, 0), 4, 0, 6, left); ar_pa.start()
    l2a.wait_recv()
    acc_l_sub(c1, 2, 0)
    al_sa = rdma_ls(sub_rows_l(c1, 0), 3, 0, 4, left); al_sa.start()
    al_pa = rdma_ls(sub_rows_l(c1, 0), 4, 0, 6, right); al_pa.start()

    r2b.wait_recv()
    acc_r_sub(c2, 2, 1)
    ar_sb = rdma_rs(sub_rows_r(c2, 1), 3, 1, 5, right); ar_sb.start()
    ar_pb = rdma_rs(sub_rows_r(c2, 1), 4, 1, 7, left); ar_pb.start()
    l2b.wait_recv()
    acc_l_sub(c1, 2, 1)
    al_sb = rdma_ls(sub_rows_l(c1, 1), 3, 1, 5, left); al_sb.start()
    al_pb = rdma_ls(sub_rows_l(c1, 1), 4, 1, 7, right); al_pb.start()

    out_ref[pl.ds(c2 * CHUNK, CHUNK), pl.ds(0, HALF)] = part_ref[
        pl.ds(c2 * CHUNK, CHUNK), pl.ds(0, HALF)].astype(jnp.float32)
    out_ref[pl.ds(c1 * CHUNK, CHUNK), pl.ds(HALF, HALF)] = part_ref[
        pl.ds(c1 * CHUNK, CHUNK), pl.ds(HALF, HALF)].astype(jnp.float32)

    ar_sa.wait_recv()
    arf_a = rdma_rs(comm_r.at[3, pl.ds(0, QB), :], 5, 0, 8, right)
    arf_a.start()
    al_sa.wait_recv()
    alf_a = rdma_ls(comm_l.at[3, pl.ds(0, QB), :], 5, 0, 8, left)
    alf_a.start()
    ar_sb.wait_recv()
    arf_b = rdma_rs(comm_r.at[3, pl.ds(QB, QB), :], 5, 1, 9, right)
    arf_b.start()
    al_sb.wait_recv()
    alf_b = rdma_ls(comm_l.at[3, pl.ds(QB, QB), :], 5, 1, 9, left)
    alf_b.start()

    out_ref[pl.ds(c0 * CHUNK, CHUNK), pl.ds(0, HALF)] = (
        comm_r[3].astype(jnp.float32))
    out_ref[pl.ds(c0 * CHUNK, CHUNK), pl.ds(HALF, HALF)] = (
        comm_l[3].astype(jnp.float32))

    ar_pa.wait_recv()
    ar_pb.wait_recv()
    out_ref[pl.ds(c3 * CHUNK, CHUNK), pl.ds(0, HALF)] = (
        comm_r[4].astype(jnp.float32))
    al_pa.wait_recv()
    al_pb.wait_recv()
    out_ref[pl.ds(c3 * CHUNK, CHUNK), pl.ds(HALF, HALF)] = (
        comm_l[4].astype(jnp.float32))

    arf_a.wait_recv()
    arf_b.wait_recv()
    out_ref[pl.ds(c1 * CHUNK, CHUNK), pl.ds(0, HALF)] = (
        comm_r[5].astype(jnp.float32))
    alf_a.wait_recv()
    alf_b.wait_recv()
    out_ref[pl.ds(c2 * CHUNK, CHUNK), pl.ds(HALF, HALF)] = (
        comm_l[5].astype(jnp.float32))

    for r in (r0, r1, l0, l1, r2a, r2b, l2a, l2b,
              ar_sa, ar_sb, ar_pa, ar_pb, al_sa, al_sb, al_pa, al_pb,
              arf_a, arf_b, alf_a, alf_b):
        r.wait_send()


def kernel(x, Wq, K_ext, V_ext, Wo):
    out = pl.pallas_call(
        _body,
        out_shape=jax.ShapeDtypeStruct((SQ, D_MODEL), jnp.float32),
        in_specs=[
            pl.BlockSpec(memory_space=pltpu.VMEM),
            pl.BlockSpec(memory_space=pl.ANY),
            pl.BlockSpec(memory_space=pltpu.VMEM),
            pl.BlockSpec(memory_space=pltpu.VMEM),
            pl.BlockSpec(memory_space=pl.ANY),
        ],
        out_specs=pl.BlockSpec(memory_space=pltpu.VMEM),
        scratch_shapes=[
            pltpu.VMEM((SQ, D_MODEL), BF),
            pltpu.VMEM((QB, HQ * DH), BF),
            pltpu.VMEM((D_MODEL, HQ * DH), BF),
            pltpu.VMEM((HQ * DH, D_MODEL), BF),
            pltpu.VMEM((D_MODEL, D_MODEL), jnp.float32),
            pltpu.VMEM((NSTEPS, CHUNK, HALF), BF),
            pltpu.VMEM((NSTEPS, CHUNK, HALF), BF),
            pltpu.SemaphoreType.DMA((NSEM,)),
            pltpu.SemaphoreType.DMA((NSEM,)),
            pltpu.SemaphoreType.DMA((NSEM,)),
            pltpu.SemaphoreType.DMA((NSEM,)),
            pltpu.SemaphoreType.DMA,
        ],
        compiler_params=pltpu.CompilerParams(
            collective_id=0, vmem_limit_bytes=100 * 1024 * 1024),
    )(x[0].astype(BF), Wq,
      K_ext[0].reshape(SQ, HQ * DH).astype(BF),
      V_ext[0].reshape(SQ, HQ * DH).astype(BF), Wo)
    return out[None]
